# Initial kernel scaffold; baseline (speedup 1.0000x reference)
import jax
import jax.numpy as jnp
from jax.experimental import pallas as pl


def kernel(x, edge_index, edge_type, bases, comp, w_self):
    raise NotImplementedError("write your pallas kernel here")


# speedup vs baseline: 9.5264x; 9.5264x over previous
import functools

import jax
import jax.numpy as jnp
from jax import lax
from jax.experimental import pallas as pl
from jax.experimental.pallas import tpu as pltpu
from jax.experimental.pallas import tpu_sc as plsc

N = 10000
E = 320000
D = 128
R = 474
B = 30

NC = 2
NS = 16
NW = NC * NS
EW = E // NW
C = 125
NCH = EW // C
PB = 32
RP = 512
NSL = N // NS
ZR = 125
NZ = NSL // ZR
DW = 16
ET = 256
GB = 5
NG = B // GB


def _sc_mesh():
    return plsc.VectorSubcoreMesh(
        core_axis_name="c", subcore_axis_name="s", num_cores=NC,
        num_subcores=NS)


def _gather_body(x_hbm, src3_hbm, xs_hbm, srcv, xbuf):
    c = lax.axis_index("c")
    s = lax.axis_index("s")
    w = c * NS + s
    pltpu.sync_copy(src3_hbm.at[w], srcv)
    base = w * NCH

    def chunk(ci, carry):
        pltpu.sync_copy(x_hbm.at[srcv.at[ci]], xbuf)
        pltpu.sync_copy(xbuf, xs_hbm.at[base + ci])
        return carry

    lax.fori_loop(0, NCH, chunk, 0)


def _sc_gather(x, src3):
    f = pl.kernel(
        _gather_body,
        out_type=jax.ShapeDtypeStruct((E // C, C, D), jnp.float32),
        mesh=_sc_mesh(),
        scratch_types=[
            pltpu.VMEM((NCH, C), jnp.int32),
            pltpu.VMEM((C, D), jnp.float32),
        ],
    )
    return f(x, src3)


def _contract_body(xs_ref, et_ref, bf_ref, cp_ref, out_ref):
    xs = xs_ref[...]
    et = et_ref[0]
    rid = lax.broadcasted_iota(jnp.int32, (ET, RP), 1)
    oh = (et == rid).astype(jnp.float32)
    cf = jnp.dot(oh, cp_ref[...], preferred_element_type=jnp.float32)
    acc = jnp.zeros((ET, D), jnp.float32)
    for g in range(NG):
        hs = jnp.dot(xs, bf_ref[:, g * GB * D:(g + 1) * GB * D],
                     preferred_element_type=jnp.float32)
        for j in range(GB):
            b = g * GB + j
            acc = acc + cf[:, b][:, None] * hs[:, j * D:(j + 1) * D]
    out_ref[...] = acc


def _tc_contract(xs, et3c, bflat, comp_pad):
    return pl.pallas_call(
        _contract_body,
        grid=(E // ET,),
        in_specs=[
            pl.BlockSpec((ET, D), lambda i: (i, 0)),
            pl.BlockSpec((1, ET, 1), lambda i: (i, 0, 0)),
            pl.BlockSpec((D, B * D), lambda i: (0, 0)),
            pl.BlockSpec((RP, PB), lambda i: (0, 0)),
        ],
        out_specs=pl.BlockSpec((ET, D), lambda i: (i, 0)),
        out_shape=jax.ShapeDtypeStruct((E, D), jnp.float32),
    )(xs, et3c, bflat, comp_pad)


def _scatter_body(msg_hbm, dst3_hbm, zrow_hbm, ones_hbm,
                  aggs_hbm, degs_hbm, dstv, msgv, agg_sh):
    c = lax.axis_index("c")
    s = lax.axis_index("s")
    w = c * NS + s
    base = w * NCH
    pltpu.sync_copy(dst3_hbm.at[w], dstv)
    pltpu.sync_copy(zrow_hbm, msgv)
    for k in range(NZ):
        pltpu.sync_copy(msgv, agg_sh.at[pl.ds(s * NSL + k * ZR, ZR)])
    plsc.subcore_barrier()

    def chunk(ci, carry):
        pltpu.sync_copy(msg_hbm.at[base + ci], msgv)
        pltpu.sync_copy(msgv, agg_sh.at[dstv.at[ci]], add=True)
        return carry

    lax.fori_loop(0, NCH, chunk, 0)
    plsc.subcore_barrier()
    for k in range(NZ):
        pltpu.sync_copy(agg_sh.at[pl.ds(s * NSL + k * ZR, ZR)], msgv)
        pltpu.sync_copy(msgv, aggs_hbm.at[c, s, k])
    plsc.subcore_barrier()

    pltpu.sync_copy(zrow_hbm, msgv)
    for k in range(NZ):
        pltpu.sync_copy(msgv, agg_sh.at[pl.ds(s * NSL + k * ZR, ZR)])
    plsc.subcore_barrier()
    pltpu.sync_copy(ones_hbm, msgv)

    def dchunk(ci, carry):
        pltpu.sync_copy(msgv, agg_sh.at[dstv.at[ci]], add=True)
        return carry

    lax.fori_loop(0, NCH, dchunk, 0)
    plsc.subcore_barrier()
    for k in range(NZ):
        pltpu.sync_copy(agg_sh.at[pl.ds(s * NSL + k * ZR, ZR)], msgv)
        pltpu.sync_copy(msgv, degs_hbm.at[c, s, k])


def _sc_scatter(msg, dst3, zrow, ones1):
    f = pl.kernel(
        _scatter_body,
        out_type=(
            jax.ShapeDtypeStruct((NC, NS, NZ, ZR, D), jnp.float32),
            jax.ShapeDtypeStruct((NC, NS, NZ, ZR, D), jnp.float32),
        ),
        mesh=_sc_mesh(),
        scratch_types=[
            pltpu.VMEM((NCH, C), jnp.int32),
            pltpu.VMEM((C, D), jnp.float32),
            pltpu.VMEM_SHARED((N, D), jnp.float32),
        ],
    )
    return f(msg, dst3, zrow, ones1)


def _finish_body(x_ref, agg_ref, deg_ref, w_ref, out_ref):
    agg = agg_ref[0] + agg_ref[1]
    deg = deg_ref[0, :, 0:1] + deg_ref[1, :, 0:1]
    norm = 1.0 / jnp.maximum(deg, 1.0)
    self_loop = jnp.dot(x_ref[...], w_ref[...],
                        preferred_element_type=jnp.float32)
    out_ref[...] = jnp.maximum(agg * norm + self_loop, 0.0)


def _tc_finish(x, aggs, degs, w_self):
    NT = 2000
    return pl.pallas_call(
        _finish_body,
        grid=(N // NT,),
        in_specs=[
            pl.BlockSpec((NT, D), lambda i: (i, 0)),
            pl.BlockSpec((NC, NT, D), lambda i: (0, i, 0)),
            pl.BlockSpec((NC, NT, D), lambda i: (0, i, 0)),
            pl.BlockSpec((D, D), lambda i: (0, 0)),
        ],
        out_specs=pl.BlockSpec((NT, D), lambda i: (i, 0)),
        out_shape=jax.ShapeDtypeStruct((N, D), jnp.float32),
    )(x, aggs, degs, w_self)


@jax.jit
def kernel(x, edge_index, edge_type, bases, comp, w_self):
    src3 = edge_index[0].reshape(NW, NCH, C)
    dst3 = edge_index[1].reshape(NW, NCH, C)
    et3c = edge_type.reshape(E // ET, ET, 1)
    comp_pad = jnp.pad(comp, ((0, RP - R), (0, PB - B)))
    bflat = jnp.transpose(bases, (1, 0, 2)).reshape(D, B * D)
    zrow = jnp.zeros((ZR, D), jnp.float32)
    ones1 = jnp.ones((C, D), jnp.float32)

    xs = _sc_gather(x, src3)
    msg = _tc_contract(xs.reshape(E, D), et3c, bflat, comp_pad)
    aggs, degs = _sc_scatter(msg.reshape(E // C, C, D), dst3, zrow, ones1)
    return _tc_finish(x, aggs.reshape(NC, N, D), degs.reshape(NC, N, D),
                      w_self)
